# baseline (speedup 1.0000x reference)
import functools

import jax
import jax.numpy as jnp
from jax import lax
from jax.experimental import pallas as pl
from jax.experimental.pallas import tpu as pltpu
from jax.experimental.pallas import tpu_sc as plsc

N = 10000
D = 128
NCLS = 64
E = 320000

NC = 2
NS = 16
NW = NC * NS

CHUNK = 128
CHUNKS = 80
NBUF = 2
EPT = CHUNKS * CHUNK
EPAD = NW * EPT
ACC_ROWS = 10240
SPARE = ACC_ROWS - N
ZROWS = ACC_ROWS // NS

_MESH = plsc.VectorSubcoreMesh(core_axis_name="c", subcore_axis_name="s")


@functools.partial(
    pl.kernel,
    out_type=jax.ShapeDtypeStruct((NC, ACC_ROWS, D), jnp.float32),
    mesh=_MESH,
    scratch_types=[
        pltpu.VMEM_SHARED((ACC_ROWS, D), jnp.float32),
        pltpu.VMEM((CHUNKS, CHUNK), jnp.int32),
        pltpu.VMEM((CHUNK, D), jnp.float32),
        pltpu.SemaphoreType.DMA,
    ],
)
def _sc_degree(dst_hbm, zeros_hbm, ones_hbm, out_hbm, acc, dst_all, ones_v,
               ssem):
    c = lax.axis_index("c")
    s = lax.axis_index("s")
    w = c * NS + s
    pltpu.sync_copy(zeros_hbm, acc.at[pl.ds(s * ZROWS, ZROWS)])
    pltpu.sync_copy(ones_hbm, ones_v)
    pltpu.sync_copy(dst_hbm.at[w], dst_all)
    plsc.subcore_barrier()

    def body(jj, carry):
        j0 = jj * NBUF
        ss = [
            pltpu.async_copy(ones_v, acc.at[dst_all.at[j0 + k]], ssem,
                             add=True)
            for k in range(NBUF)
        ]
        for d in ss:
            d.wait()
        return carry

    lax.fori_loop(0, CHUNKS // NBUF, body, 0)
    plsc.subcore_barrier()
    pltpu.sync_copy(
        acc.at[pl.ds(s * ZROWS, ZROWS)],
        out_hbm.at[c, pl.ds(s * ZROWS, ZROWS)],
    )


@functools.partial(
    pl.kernel,
    out_type=jax.ShapeDtypeStruct((NC, ACC_ROWS, D), jnp.float32),
    mesh=_MESH,
    scratch_types=[
        pltpu.VMEM_SHARED((ACC_ROWS, D), jnp.float32),
        pltpu.VMEM((CHUNKS, CHUNK), jnp.int32),
        pltpu.VMEM((CHUNK,), jnp.int32),
        pltpu.VMEM((CHUNK,), jnp.int32),
        pltpu.VMEM((CHUNK, D), jnp.float32),
        pltpu.VMEM((CHUNK, D), jnp.float32),
        pltpu.SemaphoreType.DMA,
        pltpu.SemaphoreType.DMA,
        pltpu.SemaphoreType.DMA,
        pltpu.SemaphoreType.DMA,
    ],
)
def _sc_prop(hs_hbm, src_hbm, dst_hbm, zeros_hbm, out_hbm,
             acc, dst_all, s0, s1, b0, b1, i0sem, i1sem, gsem, ssem):
    c = lax.axis_index("c")
    s = lax.axis_index("s")
    w = c * NS + s
    pltpu.sync_copy(zeros_hbm, acc.at[pl.ds(s * ZROWS, ZROWS)])
    pltpu.sync_copy(dst_hbm.at[w], dst_all)
    pltpu.async_copy(src_hbm.at[w, 0], s0, i0sem)
    plsc.subcore_barrier()

    def body(jj, carry):
        j0 = 2 * jj
        j1 = 2 * jj + 1
        jn = jnp.minimum(2 * jj + 2, CHUNKS - 1)
        pltpu.make_async_copy(src_hbm.at[w, j0], s0, i0sem).wait()
        g0 = pltpu.async_copy(hs_hbm.at[s0], b0, gsem)
        i1 = pltpu.async_copy(src_hbm.at[w, j1], s1, i1sem)
        g0.wait()
        sc0 = pltpu.async_copy(b0, acc.at[dst_all.at[j0]], ssem, add=True)
        i1.wait()
        g1 = pltpu.async_copy(hs_hbm.at[s1], b1, gsem)
        pltpu.async_copy(src_hbm.at[w, jn], s0, i0sem)
        g1.wait()
        sc1 = pltpu.async_copy(b1, acc.at[dst_all.at[j1]], ssem, add=True)
        sc0.wait()
        sc1.wait()
        return carry

    lax.fori_loop(0, CHUNKS // 2, body, 0)
    pltpu.make_async_copy(src_hbm.at[w, 0], s0, i0sem).wait()
    plsc.subcore_barrier()
    pltpu.sync_copy(
        acc.at[pl.ds(s * ZROWS, ZROWS)],
        out_hbm.at[c, pl.ds(s * ZROWS, ZROWS)],
    )


_R = 1000


def _tc_mm1_body(x, w1, h):
    h[...] = jnp.dot(x[...], w1[...], preferred_element_type=jnp.float32)


def _tc_mm1(x, W1):
    return pl.pallas_call(
        _tc_mm1_body,
        grid=(N // _R,),
        in_specs=[
            pl.BlockSpec((_R, D), lambda i: (i, 0)),
            pl.BlockSpec((D, D), lambda i: (0, 0)),
        ],
        out_specs=pl.BlockSpec((_R, D), lambda i: (i, 0)),
        out_shape=jax.ShapeDtypeStruct((N, D), jnp.float32),
    )(x, W1)


def _tc_scale_body(deg0, deg1, h1, dis, hs):
    d = lax.rsqrt(deg0[0, :, 0:1] + deg1[0, :, 0:1] + 1.0)
    dis[...] = d
    hs[...] = d * h1[...]


def _tc_scale(deg, h1):
    return pl.pallas_call(
        _tc_scale_body,
        grid=(N // _R,),
        in_specs=[
            pl.BlockSpec((1, _R, D), lambda i: (0, i, 0)),
            pl.BlockSpec((1, _R, D), lambda i: (1, i, 0)),
            pl.BlockSpec((_R, D), lambda i: (i, 0)),
        ],
        out_specs=[
            pl.BlockSpec((_R, 1), lambda i: (i, 0)),
            pl.BlockSpec((_R, D), lambda i: (i, 0)),
        ],
        out_shape=[
            jax.ShapeDtypeStruct((N, 1), jnp.float32),
            jax.ShapeDtypeStruct((N, D), jnp.float32),
        ],
    )(deg, deg, h1)


def _tc_mid_body(acc0, acc1, h1, dis, b1, w2, h2, hs2):
    d = dis[...]
    u = d * (acc0[0] + acc1[0]) + (d * d) * h1[...] + b1[...]
    u = jnp.maximum(u, 0.0)
    hh = jnp.dot(u, w2[...], preferred_element_type=jnp.float32)
    h2[...] = hh
    hs2[...] = d * hh


def _tc_mid(acc, h1, dis, b1, W2):
    return pl.pallas_call(
        _tc_mid_body,
        grid=(N // _R,),
        in_specs=[
            pl.BlockSpec((1, _R, D), lambda i: (0, i, 0)),
            pl.BlockSpec((1, _R, D), lambda i: (1, i, 0)),
            pl.BlockSpec((_R, D), lambda i: (i, 0)),
            pl.BlockSpec((_R, 1), lambda i: (i, 0)),
            pl.BlockSpec((1, D), lambda i: (0, 0)),
            pl.BlockSpec((D, D), lambda i: (0, 0)),
        ],
        out_specs=[
            pl.BlockSpec((_R, D), lambda i: (i, 0)),
            pl.BlockSpec((_R, D), lambda i: (i, 0)),
        ],
        out_shape=[
            jax.ShapeDtypeStruct((N, D), jnp.float32),
            jax.ShapeDtypeStruct((N, D), jnp.float32),
        ],
    )(acc, acc, h1, dis, b1, W2)


def _tc_fin_body(acc0, acc1, h2, dis, b2, wfc, bfc, out):
    d = dis[...]
    u = d * (acc0[0] + acc1[0]) + (d * d) * h2[...] + b2[...]
    u = jnp.maximum(u, 0.0)
    logits = jnp.dot(u, wfc[...], preferred_element_type=jnp.float32)
    logits = logits + bfc[...]
    m = jnp.max(logits, axis=1, keepdims=True)
    e = jnp.exp(logits - m)
    out[...] = e / jnp.sum(e, axis=1, keepdims=True)


def _tc_fin(acc, h2, dis, b2, Wfc, bfc):
    return pl.pallas_call(
        _tc_fin_body,
        grid=(N // _R,),
        in_specs=[
            pl.BlockSpec((1, _R, D), lambda i: (0, i, 0)),
            pl.BlockSpec((1, _R, D), lambda i: (1, i, 0)),
            pl.BlockSpec((_R, D), lambda i: (i, 0)),
            pl.BlockSpec((_R, 1), lambda i: (i, 0)),
            pl.BlockSpec((1, D), lambda i: (0, 0)),
            pl.BlockSpec((D, NCLS), lambda i: (0, 0)),
            pl.BlockSpec((1, NCLS), lambda i: (0, 0)),
        ],
        out_specs=pl.BlockSpec((_R, NCLS), lambda i: (i, 0)),
        out_shape=jax.ShapeDtypeStruct((N, NCLS), jnp.float32),
    )(acc, acc, h2, dis, b2, Wfc, bfc)


def kernel(x, edge_index, W1, b1, W2, b2, Wfc, bfc):
    src = edge_index[0].astype(jnp.int32)
    dst = edge_index[1].astype(jnp.int32)
    pad = EPAD - E
    srcp = jnp.concatenate([src, jnp.zeros((pad,), jnp.int32)])
    pad_dst = N + jnp.arange(pad, dtype=jnp.int32) % SPARE
    dstp = jnp.concatenate([dst, pad_dst])
    srcg = srcp.reshape(NW, CHUNKS, CHUNK)
    dstg = dstp.reshape(NW, CHUNKS, CHUNK)

    zeros_d = jnp.zeros((ZROWS, D), jnp.float32)
    ones_d = jnp.ones((CHUNK, D), jnp.float32)

    h1 = _tc_mm1(x, W1)
    deg = _sc_degree(dstg, zeros_d, ones_d)
    dis, hs1 = _tc_scale(deg, h1)
    acc1 = _sc_prop(hs1, srcg, dstg, zeros_d)
    h2, hs2 = _tc_mid(acc1, h1, dis, b1.reshape(1, D), W2)
    acc2 = _sc_prop(hs2, srcg, dstg, zeros_d)
    out = _tc_fin(acc2, h2, dis, b2.reshape(1, D), Wfc, bfc.reshape(1, NCLS))
    return out


# speedup vs baseline: 1.2229x; 1.2229x over previous
import functools

import jax
import jax.numpy as jnp
from jax import lax
from jax.experimental import pallas as pl
from jax.experimental.pallas import tpu as pltpu
from jax.experimental.pallas import tpu_sc as plsc

N = 10000
D = 128
NCLS = 64
E = 320000

NC = 2
NS = 16
NW = NC * NS

CHUNK = 128
CH_A = 146
CH_B = 14
NCHT = NS * (CH_A + CH_B)
EPAD = NCHT * CHUNK
ACC_ROWS = 10240
SPARE = ACC_ROWS - N
ZROWS = ACC_ROWS // NS

_MESH = plsc.VectorSubcoreMesh(core_axis_name="c", subcore_axis_name="s")


@functools.partial(
    pl.kernel,
    out_type=jax.ShapeDtypeStruct((NC, ACC_ROWS, D), jnp.float32),
    mesh=_MESH,
    scratch_types=[
        pltpu.VMEM_SHARED((ACC_ROWS, D), jnp.float32),
        pltpu.VMEM((80, CHUNK), jnp.int32),
        pltpu.VMEM((CHUNK, D), jnp.float32),
        pltpu.SemaphoreType.DMA,
    ],
)
def _sc_degree(dst_hbm, zeros_hbm, ones_hbm, out_hbm, acc, dst_all, ones_v,
               ssem):
    c = lax.axis_index("c")
    s = lax.axis_index("s")
    w = c * NS + s
    pltpu.sync_copy(zeros_hbm, acc.at[pl.ds(s * ZROWS, ZROWS)])
    pltpu.sync_copy(ones_hbm, ones_v)
    pltpu.sync_copy(dst_hbm.at[pl.ds(w * 80, 80)], dst_all)
    plsc.subcore_barrier()

    def body(jj, carry):
        s0 = pltpu.async_copy(ones_v, acc.at[dst_all.at[2 * jj]], ssem,
                              add=True)
        s1 = pltpu.async_copy(ones_v, acc.at[dst_all.at[2 * jj + 1]], ssem,
                              add=True)
        s0.wait()
        s1.wait()
        return carry

    lax.fori_loop(0, 40, body, 0)
    plsc.subcore_barrier()
    pltpu.sync_copy(
        acc.at[pl.ds(s * ZROWS, ZROWS)],
        out_hbm.at[c, pl.ds(s * ZROWS, ZROWS)],
    )


@functools.partial(
    pl.kernel,
    out_type=jax.ShapeDtypeStruct((NC, ACC_ROWS, D), jnp.float32),
    mesh=_MESH,
    scratch_types=[
        pltpu.VMEM_SHARED((ACC_ROWS, D), jnp.float32),
        pltpu.VMEM((CHUNK,), jnp.int32),
        pltpu.VMEM((CHUNK,), jnp.int32),
        pltpu.VMEM((CHUNK,), jnp.int32),
        pltpu.VMEM((CHUNK,), jnp.int32),
        pltpu.VMEM((CHUNK, D), jnp.float32),
        pltpu.VMEM((CHUNK, D), jnp.float32),
        pltpu.SemaphoreType.DMA,
        pltpu.SemaphoreType.DMA,
        pltpu.SemaphoreType.DMA,
        pltpu.SemaphoreType.DMA,
        pltpu.SemaphoreType.DMA,
        pltpu.SemaphoreType.DMA,
    ],
)
def _sc_prop(hs_hbm, src_hbm, dst_hbm, zeros_hbm, out_hbm,
             acc, s0, s1, d0, d1, b0, b1, i0sem, i1sem, j0sem, j1sem,
             gsem, ssem):
    c = lax.axis_index("c")
    s = lax.axis_index("s")
    nch = jnp.where(c == 0, CH_A, CH_B)
    base = jnp.where(c == 0, s * CH_A, NS * CH_A + s * CH_B)
    pltpu.sync_copy(zeros_hbm, acc.at[pl.ds(s * ZROWS, ZROWS)])
    pltpu.async_copy(src_hbm.at[base], s0, i0sem)
    pltpu.async_copy(dst_hbm.at[base], d0, j0sem)
    plsc.subcore_barrier()

    last = base + nch - 1

    def body(jj, carry):
        j0 = base + 2 * jj
        j1 = j0 + 1
        jn = jnp.minimum(j0 + 2, last)
        pltpu.make_async_copy(src_hbm.at[j0], s0, i0sem).wait()
        pltpu.make_async_copy(dst_hbm.at[j0], d0, j0sem).wait()
        g0 = pltpu.async_copy(hs_hbm.at[s0], b0, gsem)
        i1 = pltpu.async_copy(src_hbm.at[j1], s1, i1sem)
        i1b = pltpu.async_copy(dst_hbm.at[j1], d1, j1sem)
        g0.wait()
        sc0 = pltpu.async_copy(b0, acc.at[d0], ssem, add=True)
        i1.wait()
        i1b.wait()
        g1 = pltpu.async_copy(hs_hbm.at[s1], b1, gsem)
        g1.wait()
        sc1 = pltpu.async_copy(b1, acc.at[d1], ssem, add=True)
        sc0.wait()
        pltpu.async_copy(src_hbm.at[jn], s0, i0sem)
        pltpu.async_copy(dst_hbm.at[jn], d0, j0sem)
        sc1.wait()
        return carry

    lax.fori_loop(0, nch // 2, body, 0)
    pltpu.make_async_copy(src_hbm.at[base], s0, i0sem).wait()
    pltpu.make_async_copy(dst_hbm.at[base], d0, j0sem).wait()

    plsc.subcore_barrier()
    pltpu.sync_copy(
        acc.at[pl.ds(s * ZROWS, ZROWS)],
        out_hbm.at[c, pl.ds(s * ZROWS, ZROWS)],
    )


_R = 1000


def _tc_mm1_body(x, w1, h):
    h[...] = jnp.dot(x[...], w1[...], preferred_element_type=jnp.float32)


def _tc_mm1(x, W1):
    return pl.pallas_call(
        _tc_mm1_body,
        grid=(N // _R,),
        in_specs=[
            pl.BlockSpec((_R, D), lambda i: (i, 0)),
            pl.BlockSpec((D, D), lambda i: (0, 0)),
        ],
        out_specs=pl.BlockSpec((_R, D), lambda i: (i, 0)),
        out_shape=jax.ShapeDtypeStruct((N, D), jnp.float32),
    )(x, W1)


def _tc_scale_body(deg0, deg1, h1, dis, hs):
    d = lax.rsqrt(deg0[0] + deg1[0] + 1.0)
    dis[...] = d
    hs[...] = d * h1[...]


def _tc_scale(deg, h1):
    return pl.pallas_call(
        _tc_scale_body,
        grid=(N // _R,),
        in_specs=[
            pl.BlockSpec((1, _R, 1), lambda i: (0, i, 0)),
            pl.BlockSpec((1, _R, 1), lambda i: (1, i, 0)),
            pl.BlockSpec((_R, D), lambda i: (i, 0)),
        ],
        out_specs=[
            pl.BlockSpec((_R, 1), lambda i: (i, 0)),
            pl.BlockSpec((_R, D), lambda i: (i, 0)),
        ],
        out_shape=[
            jax.ShapeDtypeStruct((N, 1), jnp.float32),
            jax.ShapeDtypeStruct((N, D), jnp.float32),
        ],
    )(deg, deg, h1)


def _tc_mid_body(acc0, acc1, h1, dis, b1, w2, h2, hs2):
    d = dis[...]
    u = d * (acc0[0] + acc1[0]) + (d * d) * h1[...] + b1[...]
    u = jnp.maximum(u, 0.0)
    hh = jnp.dot(u, w2[...], preferred_element_type=jnp.float32)
    h2[...] = hh
    hs2[...] = d * hh


def _tc_mid(acc, h1, dis, b1, W2):
    return pl.pallas_call(
        _tc_mid_body,
        grid=(N // _R,),
        in_specs=[
            pl.BlockSpec((1, _R, D), lambda i: (0, i, 0)),
            pl.BlockSpec((1, _R, D), lambda i: (1, i, 0)),
            pl.BlockSpec((_R, D), lambda i: (i, 0)),
            pl.BlockSpec((_R, 1), lambda i: (i, 0)),
            pl.BlockSpec((1, D), lambda i: (0, 0)),
            pl.BlockSpec((D, D), lambda i: (0, 0)),
        ],
        out_specs=[
            pl.BlockSpec((_R, D), lambda i: (i, 0)),
            pl.BlockSpec((_R, D), lambda i: (i, 0)),
        ],
        out_shape=[
            jax.ShapeDtypeStruct((N, D), jnp.float32),
            jax.ShapeDtypeStruct((N, D), jnp.float32),
        ],
    )(acc, acc, h1, dis, b1, W2)


def _tc_fin_body(acc0, acc1, h2, dis, b2, wfc, bfc, out):
    d = dis[...]
    u = d * (acc0[0] + acc1[0]) + (d * d) * h2[...] + b2[...]
    u = jnp.maximum(u, 0.0)
    logits = jnp.dot(u, wfc[...], preferred_element_type=jnp.float32)
    logits = logits + bfc[...]
    m = jnp.max(logits, axis=1, keepdims=True)
    e = jnp.exp(logits - m)
    out[...] = e / jnp.sum(e, axis=1, keepdims=True)


def _tc_fin(acc, h2, dis, b2, Wfc, bfc):
    return pl.pallas_call(
        _tc_fin_body,
        grid=(N // _R,),
        in_specs=[
            pl.BlockSpec((1, _R, D), lambda i: (0, i, 0)),
            pl.BlockSpec((1, _R, D), lambda i: (1, i, 0)),
            pl.BlockSpec((_R, D), lambda i: (i, 0)),
            pl.BlockSpec((_R, 1), lambda i: (i, 0)),
            pl.BlockSpec((1, D), lambda i: (0, 0)),
            pl.BlockSpec((D, NCLS), lambda i: (0, 0)),
            pl.BlockSpec((1, NCLS), lambda i: (0, 0)),
        ],
        out_specs=pl.BlockSpec((_R, NCLS), lambda i: (i, 0)),
        out_shape=jax.ShapeDtypeStruct((N, NCLS), jnp.float32),
    )(acc, acc, h2, dis, b2, Wfc, bfc)


def kernel(x, edge_index, W1, b1, W2, b2, Wfc, bfc):
    src = edge_index[0].astype(jnp.int32)
    dst = edge_index[1].astype(jnp.int32)
    pad = EPAD - E
    srcp = jnp.concatenate([src, jnp.zeros((pad,), jnp.int32)])
    pad_dst = N + jnp.arange(pad, dtype=jnp.int32) % SPARE
    dstp = jnp.concatenate([dst, pad_dst])
    srcg = srcp.reshape(NCHT, CHUNK)
    dstg = dstp.reshape(NCHT, CHUNK)

    zeros_d = jnp.zeros((ZROWS, D), jnp.float32)
    ones_d = jnp.ones((CHUNK, D), jnp.float32)

    h1 = _tc_mm1(x, W1)
    degw = _sc_degree(dstg, zeros_d, ones_d)
    deg = degw[:, :, 0:1]
    dis, hs1 = _tc_scale(deg, h1)
    acc1 = _sc_prop(hs1, srcg, dstg, zeros_d)
    h2, hs2 = _tc_mid(acc1, h1, dis, b1.reshape(1, D), W2)
    acc2 = _sc_prop(hs2, srcg, dstg, zeros_d)
    out = _tc_fin(acc2, h2, dis, b2.reshape(1, D), Wfc, bfc.reshape(1, NCLS))
    return out


# speedup vs baseline: 1.2290x; 1.0050x over previous
import functools

import jax
import jax.numpy as jnp
from jax import lax
from jax.experimental import pallas as pl
from jax.experimental.pallas import tpu as pltpu
from jax.experimental.pallas import tpu_sc as plsc

N = 10000
D = 128
NCLS = 64
E = 320000

NC = 2
NS = 16
NW = NC * NS

CHUNK = 128
CH_A = 148
CH_B = 12
NCHT = NS * (CH_A + CH_B)
EPAD = NCHT * CHUNK
ACC_ROWS = 10240
SPARE = ACC_ROWS - N
ZROWS = ACC_ROWS // NS

_MESH = plsc.VectorSubcoreMesh(core_axis_name="c", subcore_axis_name="s")


@functools.partial(
    pl.kernel,
    out_type=jax.ShapeDtypeStruct((NC, ACC_ROWS, D), jnp.float32),
    mesh=_MESH,
    scratch_types=[
        pltpu.VMEM_SHARED((ACC_ROWS, D), jnp.float32),
        pltpu.VMEM((80, CHUNK), jnp.int32),
        pltpu.VMEM((CHUNK, D), jnp.float32),
        pltpu.SemaphoreType.DMA,
    ],
)
def _sc_degree(dst_hbm, zeros_hbm, ones_hbm, out_hbm, acc, dst_all, ones_v,
               ssem):
    c = lax.axis_index("c")
    s = lax.axis_index("s")
    w = c * NS + s
    pltpu.sync_copy(zeros_hbm, acc.at[pl.ds(s * ZROWS, ZROWS)])
    pltpu.sync_copy(ones_hbm, ones_v)
    pltpu.sync_copy(dst_hbm.at[pl.ds(w * 80, 80)], dst_all)
    plsc.subcore_barrier()

    def body(jj, carry):
        s0 = pltpu.async_copy(ones_v, acc.at[dst_all.at[2 * jj]], ssem,
                              add=True)
        s1 = pltpu.async_copy(ones_v, acc.at[dst_all.at[2 * jj + 1]], ssem,
                              add=True)
        s0.wait()
        s1.wait()
        return carry

    lax.fori_loop(0, 40, body, 0)
    plsc.subcore_barrier()
    pltpu.sync_copy(
        acc.at[pl.ds(s * ZROWS, ZROWS)],
        out_hbm.at[c, pl.ds(s * ZROWS, ZROWS)],
    )


@functools.partial(
    pl.kernel,
    out_type=jax.ShapeDtypeStruct((NC, ACC_ROWS, D), jnp.float32),
    mesh=_MESH,
    scratch_types=[
        pltpu.VMEM_SHARED((ACC_ROWS, D), jnp.float32),
        pltpu.VMEM((CHUNK,), jnp.int32),
        pltpu.VMEM((CHUNK,), jnp.int32),
        pltpu.VMEM((CHUNK,), jnp.int32),
        pltpu.VMEM((CHUNK,), jnp.int32),
        pltpu.VMEM((CHUNK, D), jnp.float32),
        pltpu.VMEM((CHUNK, D), jnp.float32),
        pltpu.SemaphoreType.DMA,
        pltpu.SemaphoreType.DMA,
        pltpu.SemaphoreType.DMA,
        pltpu.SemaphoreType.DMA,
        pltpu.SemaphoreType.DMA,
        pltpu.SemaphoreType.DMA,
    ],
)
def _sc_prop(hs_hbm, src_hbm, dst_hbm, zeros_hbm, out_hbm,
             acc, s0, s1, d0, d1, b0, b1, i0sem, i1sem, j0sem, j1sem,
             gsem, ssem):
    c = lax.axis_index("c")
    s = lax.axis_index("s")
    nch = jnp.where(c == 0, CH_A, CH_B)
    base = jnp.where(c == 0, s * CH_A, NS * CH_A + s * CH_B)
    pltpu.sync_copy(zeros_hbm, acc.at[pl.ds(s * ZROWS, ZROWS)])
    pltpu.async_copy(src_hbm.at[base], s0, i0sem)
    pltpu.async_copy(dst_hbm.at[base], d0, j0sem)
    plsc.subcore_barrier()

    last = base + nch - 1

    def body(jj, carry):
        j0 = base + 2 * jj
        j1 = j0 + 1
        jn = jnp.minimum(j0 + 2, last)
        pltpu.make_async_copy(src_hbm.at[j0], s0, i0sem).wait()
        pltpu.make_async_copy(dst_hbm.at[j0], d0, j0sem).wait()
        g0 = pltpu.async_copy(hs_hbm.at[s0], b0, gsem)
        i1 = pltpu.async_copy(src_hbm.at[j1], s1, i1sem)
        i1b = pltpu.async_copy(dst_hbm.at[j1], d1, j1sem)
        g0.wait()
        sc0 = pltpu.async_copy(b0, acc.at[d0], ssem, add=True)
        i1.wait()
        i1b.wait()
        g1 = pltpu.async_copy(hs_hbm.at[s1], b1, gsem)
        g1.wait()
        sc1 = pltpu.async_copy(b1, acc.at[d1], ssem, add=True)
        sc0.wait()
        pltpu.async_copy(src_hbm.at[jn], s0, i0sem)
        pltpu.async_copy(dst_hbm.at[jn], d0, j0sem)
        sc1.wait()
        return carry

    lax.fori_loop(0, nch // 2, body, 0)
    pltpu.make_async_copy(src_hbm.at[base], s0, i0sem).wait()
    pltpu.make_async_copy(dst_hbm.at[base], d0, j0sem).wait()

    plsc.subcore_barrier()
    pltpu.sync_copy(
        acc.at[pl.ds(s * ZROWS, ZROWS)],
        out_hbm.at[c, pl.ds(s * ZROWS, ZROWS)],
    )


_R = 1000


def _tc_mm1_body(x, w1, h):
    h[...] = jnp.dot(x[...], w1[...], preferred_element_type=jnp.float32)


def _tc_mm1(x, W1):
    return pl.pallas_call(
        _tc_mm1_body,
        grid=(N // _R,),
        in_specs=[
            pl.BlockSpec((_R, D), lambda i: (i, 0)),
            pl.BlockSpec((D, D), lambda i: (0, 0)),
        ],
        out_specs=pl.BlockSpec((_R, D), lambda i: (i, 0)),
        out_shape=jax.ShapeDtypeStruct((N, D), jnp.float32),
    )(x, W1)


def _tc_scale_body(deg0, deg1, h1, dis, hs):
    d = lax.rsqrt(deg0[0] + deg1[0] + 1.0)
    dis[...] = d
    hs[...] = d * h1[...]


def _tc_scale(deg, h1):
    return pl.pallas_call(
        _tc_scale_body,
        grid=(N // _R,),
        in_specs=[
            pl.BlockSpec((1, _R, 1), lambda i: (0, i, 0)),
            pl.BlockSpec((1, _R, 1), lambda i: (1, i, 0)),
            pl.BlockSpec((_R, D), lambda i: (i, 0)),
        ],
        out_specs=[
            pl.BlockSpec((_R, 1), lambda i: (i, 0)),
            pl.BlockSpec((_R, D), lambda i: (i, 0)),
        ],
        out_shape=[
            jax.ShapeDtypeStruct((N, 1), jnp.float32),
            jax.ShapeDtypeStruct((N, D), jnp.float32),
        ],
    )(deg, deg, h1)


def _tc_mid_body(acc0, acc1, h1, dis, b1, w2, h2, hs2):
    d = dis[...]
    u = d * (acc0[0] + acc1[0]) + (d * d) * h1[...] + b1[...]
    u = jnp.maximum(u, 0.0)
    hh = jnp.dot(u, w2[...], preferred_element_type=jnp.float32)
    h2[...] = hh
    hs2[...] = d * hh


def _tc_mid(acc, h1, dis, b1, W2):
    return pl.pallas_call(
        _tc_mid_body,
        grid=(N // _R,),
        in_specs=[
            pl.BlockSpec((1, _R, D), lambda i: (0, i, 0)),
            pl.BlockSpec((1, _R, D), lambda i: (1, i, 0)),
            pl.BlockSpec((_R, D), lambda i: (i, 0)),
            pl.BlockSpec((_R, 1), lambda i: (i, 0)),
            pl.BlockSpec((1, D), lambda i: (0, 0)),
            pl.BlockSpec((D, D), lambda i: (0, 0)),
        ],
        out_specs=[
            pl.BlockSpec((_R, D), lambda i: (i, 0)),
            pl.BlockSpec((_R, D), lambda i: (i, 0)),
        ],
        out_shape=[
            jax.ShapeDtypeStruct((N, D), jnp.float32),
            jax.ShapeDtypeStruct((N, D), jnp.float32),
        ],
    )(acc, acc, h1, dis, b1, W2)


def _tc_fin_body(acc0, acc1, h2, dis, b2, wfc, bfc, out):
    d = dis[...]
    u = d * (acc0[0] + acc1[0]) + (d * d) * h2[...] + b2[...]
    u = jnp.maximum(u, 0.0)
    logits = jnp.dot(u, wfc[...], preferred_element_type=jnp.float32)
    logits = logits + bfc[...]
    m = jnp.max(logits, axis=1, keepdims=True)
    e = jnp.exp(logits - m)
    out[...] = e / jnp.sum(e, axis=1, keepdims=True)


def _tc_fin(acc, h2, dis, b2, Wfc, bfc):
    return pl.pallas_call(
        _tc_fin_body,
        grid=(N // _R,),
        in_specs=[
            pl.BlockSpec((1, _R, D), lambda i: (0, i, 0)),
            pl.BlockSpec((1, _R, D), lambda i: (1, i, 0)),
            pl.BlockSpec((_R, D), lambda i: (i, 0)),
            pl.BlockSpec((_R, 1), lambda i: (i, 0)),
            pl.BlockSpec((1, D), lambda i: (0, 0)),
            pl.BlockSpec((D, NCLS), lambda i: (0, 0)),
            pl.BlockSpec((1, NCLS), lambda i: (0, 0)),
        ],
        out_specs=pl.BlockSpec((_R, NCLS), lambda i: (i, 0)),
        out_shape=jax.ShapeDtypeStruct((N, NCLS), jnp.float32),
    )(acc, acc, h2, dis, b2, Wfc, bfc)


def kernel(x, edge_index, W1, b1, W2, b2, Wfc, bfc):
    src = edge_index[0].astype(jnp.int32)
    dst = edge_index[1].astype(jnp.int32)
    pad = EPAD - E
    srcp = jnp.concatenate([src, jnp.zeros((pad,), jnp.int32)])
    pad_dst = N + jnp.arange(pad, dtype=jnp.int32) % SPARE
    dstp = jnp.concatenate([dst, pad_dst])
    srcg = srcp.reshape(NCHT, CHUNK)
    dstg = dstp.reshape(NCHT, CHUNK)

    zeros_d = jnp.zeros((ZROWS, D), jnp.float32)
    ones_d = jnp.ones((CHUNK, D), jnp.float32)

    h1 = _tc_mm1(x, W1)
    degw = _sc_degree(dstg, zeros_d, ones_d)
    deg = degw[:, :, 0:1]
    dis, hs1 = _tc_scale(deg, h1)
    acc1 = _sc_prop(hs1, srcg, dstg, zeros_d)
    h2, hs2 = _tc_mid(acc1, h1, dis, b1.reshape(1, D), W2)
    acc2 = _sc_prop(hs2, srcg, dstg, zeros_d)
    out = _tc_fin(acc2, h2, dis, b2.reshape(1, D), Wfc, bfc.reshape(1, NCLS))
    return out
